# baseline (speedup 1.0000x reference)
import functools

import numpy as np
import jax
import jax.numpy as jnp
from jax import lax
from jax.experimental import pallas as pl
from jax.experimental.pallas import tpu as pltpu
from jax.experimental.pallas import tpu_sc as plsc

_G = 128


def _chunk_table(batch_size: int, time_steps: int):
    rng = np.random.RandomState(0)
    drop_times = rng.randint(1, 10 + 1, size=batch_size)
    chunks = []
    for i in range(batch_size):
        n = int(drop_times[i])
        lengths = rng.randint(100, 1000 + 1, size=n)
        start_max = time_steps - int(lengths.max())
        ss = rng.randint(0, start_max + 1, size=n)
        merged = []
        for s, l in sorted((int(s), int(l)) for s, l in zip(ss, lengths)):
            if merged and s <= merged[-1][1]:
                merged[-1][1] = max(merged[-1][1], s + l)
            else:
                merged.append([s, s + l])
        chunks.append(merged)
    return chunks


def _granule_tables(b, t, nw):
    chunks = _chunk_table(b, t)
    rows_per_w = b // nw
    interiors = [[] for _ in range(nw)]
    edges = [dict() for _ in range(nw)]
    for r, row_chunks in enumerate(chunks):
        w = r // rows_per_w
        for s, e in row_chunks:
            lo, hi = r * t + s, r * t + e
            g0, g1 = -(-lo // _G), hi // _G
            interiors[w].extend(range(g0, g1))
            for g in {lo // _G, (hi - 1) // _G} - set(range(g0, g1)):
                m = edges[w].setdefault(g, np.ones(_G, np.float32))
                pos = g * _G + np.arange(_G)
                m[(pos >= lo) & (pos < hi)] = 0.0
    ki = max(1, -(-max(len(x) for x in interiors) // 128))
    ke = max(1, -(-max(len(x) for x in edges) // 128))
    int_idx = np.zeros((nw, ki, 128), np.int32)
    edge_idx = np.zeros((nw, ke, 128), np.int32)
    edge_mask = np.zeros((nw, ke, 128, _G), np.float32)
    for w in range(nw):
        pad = interiors[w][0]
        int_idx[w] = np.array(
            interiors[w] + [pad] * (ki * 128 - len(interiors[w])),
            np.int32).reshape(ki, 128)
        eg = sorted(edges[w].items())
        idxs = [g for g, _ in eg] + [pad] * (ke * 128 - len(eg))
        masks = [m for _, m in eg] + [np.zeros(_G, np.float32)] * (ke * 128 - len(eg))
        edge_idx[w] = np.array(idxs, np.int32).reshape(ke, 128)
        edge_mask[w] = np.stack(masks).reshape(ke, 128, _G)
    return int_idx, edge_idx, edge_mask, ki, ke


def kernel(clean_waveform, clean_len):
    del clean_len
    b, t = clean_waveform.shape
    info = plsc.get_sparse_core_info()
    nc, ns = info.num_cores, info.num_subcores
    nw = nc * ns
    assert b % nw == 0 and t % _G == 0
    nr = b * t // _G
    rpw = nr // nw
    int_idx, edge_idx, edge_mask, ki, ke = _granule_tables(b, t, nw)

    x2d = clean_waveform.reshape(nr, _G)
    zeros128 = jnp.zeros((128, _G), jnp.float32)
    mesh = plsc.VectorSubcoreMesh(core_axis_name="c", subcore_axis_name="s")

    @functools.partial(
        pl.kernel, mesh=mesh,
        out_type=jax.ShapeDtypeStruct((nr, _G), jnp.float32),
        scratch_types=[
            pltpu.VMEM((128,), jnp.int32),
            pltpu.VMEM((128, _G), jnp.float32),
            pltpu.VMEM((128, _G), jnp.float32),
            pltpu.VMEM((128, _G), jnp.float32),
        ],
    )
    def sc_drop(x_hbm, z_hbm, ii_hbm, ei_hbm, em_hbm, o_hbm,
                idx_v, zeros_v, edge_v, mask_v):
        wid = lax.axis_index("s") * nc + lax.axis_index("c")
        base = wid * rpw
        pltpu.sync_copy(x_hbm.at[pl.ds(base, rpw)], o_hbm.at[pl.ds(base, rpw)])
        pltpu.sync_copy(z_hbm, zeros_v)
        for j in range(ki):
            pltpu.sync_copy(ii_hbm.at[wid, j], idx_v)
            pltpu.sync_copy(zeros_v, o_hbm.at[idx_v])
        for j in range(ke):
            pltpu.sync_copy(ei_hbm.at[wid, j], idx_v)
            pltpu.sync_copy(o_hbm.at[idx_v], edge_v)
            pltpu.sync_copy(em_hbm.at[wid, j], mask_v)
            edge_v[...] = edge_v[...] * mask_v[...]
            pltpu.sync_copy(edge_v, o_hbm.at[idx_v])

    out = sc_drop(x2d, zeros128, jnp.asarray(int_idx),
                  jnp.asarray(edge_idx), jnp.asarray(edge_mask))
    return out.reshape(b, t)


# speedup vs baseline: 10.0618x; 10.0618x over previous
import functools

import numpy as np
import jax
import jax.numpy as jnp
from jax import lax
from jax.experimental import pallas as pl
from jax.experimental.pallas import tpu as pltpu
from jax.experimental.pallas import tpu_sc as plsc

_G = 128
_W = 32000


def _chunk_table(batch_size: int, time_steps: int):
    rng = np.random.RandomState(0)
    drop_times = rng.randint(1, 10 + 1, size=batch_size)
    chunks = []
    for i in range(batch_size):
        n = int(drop_times[i])
        lengths = rng.randint(100, 1000 + 1, size=n)
        start_max = time_steps - int(lengths.max())
        ss = rng.randint(0, start_max + 1, size=n)
        merged = []
        for s, l in sorted((int(s), int(l)) for s, l in zip(ss, lengths)):
            if merged and s <= merged[-1][1]:
                merged[-1][1] = max(merged[-1][1], s + l)
            else:
                merged.append([s, s + l])
        chunks.append(merged)
    return chunks


def _granule_tables(b, t, nw):
    chunks = _chunk_table(b, t)
    rows_per_w = b // nw
    interiors = [[] for _ in range(nw)]
    edges = [dict() for _ in range(nw)]
    for r, row_chunks in enumerate(chunks):
        w = r // rows_per_w
        for s, e in row_chunks:
            lo, hi = r * t + s, r * t + e
            g0, g1 = -(-lo // _G), hi // _G
            interiors[w].extend(range(g0, g1))
            for g in {lo // _G, (hi - 1) // _G} - set(range(g0, g1)):
                m = edges[w].setdefault(g, np.ones(_G, np.float32))
                pos = g * _G + np.arange(_G)
                m[(pos >= lo) & (pos < hi)] = 0.0
    for w in range(nw):
        assert interiors[w], "worker with no interior granule"
    ki = max(1, -(-max(len(x) for x in interiors) // 128))
    ke = max(1, -(-max(len(x) for x in edges) // 128))
    int_idx = np.zeros((nw, ki, 128), np.int32)
    edge_idx = np.zeros((nw, ke, 128), np.int32)
    edge_mask = np.zeros((nw, ke, 128, _G), np.float32)
    for w in range(nw):
        pad = interiors[w][0]
        int_idx[w] = np.array(
            interiors[w] + [pad] * (ki * 128 - len(interiors[w])),
            np.int32).reshape(ki, 128)
        eg = sorted(edges[w].items())
        idxs = [g for g, _ in eg] + [pad] * (ke * 128 - len(eg))
        masks = [m for _, m in eg] + [np.zeros(_G, np.float32)] * (ke * 128 - len(eg))
        edge_idx[w] = np.array(idxs, np.int32).reshape(ke, 128)
        edge_mask[w] = np.stack(masks).reshape(ke, 128, _G)
    return int_idx, edge_idx, edge_mask, ki, ke


def _copy_body(x_ref, o_ref):
    o_ref[...] = x_ref[...]


def _tc_copy(x):
    b, t = x.shape
    return pl.pallas_call(
        _copy_body,
        grid=(t // _W,),
        in_specs=[pl.BlockSpec((b, _W), lambda j: (0, j))],
        out_specs=pl.BlockSpec((b, _W), lambda j: (0, j)),
        out_shape=jax.ShapeDtypeStruct((b, t), x.dtype),
        compiler_params=pltpu.CompilerParams(
            dimension_semantics=("arbitrary",),
        ),
    )(x)


def kernel(clean_waveform, clean_len):
    del clean_len
    b, t = clean_waveform.shape
    info = plsc.get_sparse_core_info()
    nc, ns = info.num_cores, info.num_subcores
    nw = nc * ns
    assert b % nw == 0 and t % _G == 0 and t % _W == 0
    nr = b * t // _G
    int_idx, edge_idx, edge_mask, ki, ke = _granule_tables(b, t, nw)

    zeros128 = jnp.zeros((128, _G), jnp.float32)
    mesh = plsc.VectorSubcoreMesh(core_axis_name="c", subcore_axis_name="s")

    @functools.partial(
        pl.kernel, mesh=mesh,
        out_type=(),
        scratch_types=[
            pltpu.VMEM((128,), jnp.int32),
            pltpu.VMEM((128, _G), jnp.float32),
            pltpu.VMEM((128, _G), jnp.float32),
            pltpu.VMEM((128, _G), jnp.float32),
        ],
    )
    def sc_zero(o_hbm, z_hbm, ii_hbm, ei_hbm, em_hbm,
                idx_v, zeros_v, edge_v, mask_v):
        wid = lax.axis_index("s") * nc + lax.axis_index("c")
        pltpu.sync_copy(z_hbm, zeros_v)
        for j in range(ki):
            pltpu.sync_copy(ii_hbm.at[wid, j], idx_v)
            pltpu.sync_copy(zeros_v, o_hbm.at[idx_v])
        for j in range(ke):
            pltpu.sync_copy(ei_hbm.at[wid, j], idx_v)
            pltpu.sync_copy(o_hbm.at[idx_v], edge_v)
            pltpu.sync_copy(em_hbm.at[wid, j], mask_v)
            edge_v[...] = edge_v[...] * mask_v[...]
            pltpu.sync_copy(edge_v, o_hbm.at[idx_v])

    buf = jax.new_ref(_tc_copy(clean_waveform).reshape(nr, _G))
    sc_zero(buf, zeros128, jnp.asarray(int_idx),
            jnp.asarray(edge_idx), jnp.asarray(edge_mask))
    return buf[...].reshape(b, t)


# speedup vs baseline: 49.4023x; 4.9099x over previous
import numpy as np
import jax
import jax.numpy as jnp
from jax.experimental import pallas as pl
from jax.experimental.pallas import tpu as pltpu

_W = 32000
_WIN = 1152


def _chunk_table(batch_size: int, time_steps: int):
    rng = np.random.RandomState(0)
    drop_times = rng.randint(1, 10 + 1, size=batch_size)
    chunks = []
    for i in range(batch_size):
        n = int(drop_times[i])
        lengths = rng.randint(100, 1000 + 1, size=n)
        start_max = time_steps - int(lengths.max())
        ss = rng.randint(0, start_max + 1, size=n)
        chunks.append([(int(s), int(s) + int(l)) for s, l in zip(ss, lengths)])
    return chunks


def _entry_table(batch_size: int, time_steps: int, nb: int):
    chunks = _chunk_table(batch_size, time_steps)
    entries = [[] for _ in range(nb)]
    for r, row_chunks in enumerate(chunks):
        merged = []
        for lo, hi in sorted(row_chunks):
            if merged and lo <= merged[-1][1]:
                merged[-1][1] = max(merged[-1][1], hi)
            else:
                merged.append([lo, hi])
        for s, e in merged:
            for j in range(s // _W, (e - 1) // _W + 1):
                ls = max(s, j * _W) - j * _W
                le = min(e, (j + 1) * _W) - j * _W
                if le <= ls:
                    continue
                rbase = (r // 8) * 8
                lrow = r - rbase
                for p in range(ls, le, 1024):
                    pls, ple = p, min(p + 1024, le)
                    w = min((pls // 128) * 128, _W - _WIN)
                    entries[j].append(
                        (rbase // 8, w // 128, lrow * time_steps + (pls - w),
                         lrow * time_steps + (ple - w)))
    cnt = np.array([len(ej) for ej in entries], np.int32)
    me = max(1, int(cnt.max()))
    ent = np.zeros((nb, me, 4), np.int32)
    for j, ej in enumerate(entries):
        for k, e4 in enumerate(ej):
            ent[j, k] = e4
    return ent, cnt


def _make_body(time_steps):
    def _body(ent_ref, cnt_ref, x_ref, o_ref):
        j = pl.program_id(0)
        o_ref[...] = x_ref[...]
        pat = (jax.lax.broadcasted_iota(jnp.int32, (8, _WIN), 0) * time_steps
               + jax.lax.broadcasted_iota(jnp.int32, (8, _WIN), 1))

        def loop(k, carry):
            rbase = ent_ref[j, k, 0] * 8
            w = ent_ref[j, k, 1] * 128
            lo = ent_ref[j, k, 2]
            hi = ent_ref[j, k, 3]
            win = o_ref[pl.ds(rbase, 8), pl.ds(w, _WIN)]
            keep = (pat < lo) | (pat >= hi)
            o_ref[pl.ds(rbase, 8), pl.ds(w, _WIN)] = jnp.where(keep, win, 0.0)
            return carry

        jax.lax.fori_loop(0, cnt_ref[j], loop, 0)
    return _body


def kernel(clean_waveform, clean_len):
    del clean_len
    b, t = clean_waveform.shape
    nb = t // _W
    ent, cnt = _entry_table(b, t, nb)

    smem = pl.BlockSpec(memory_space=pltpu.SMEM)
    return pl.pallas_call(
        _make_body(t),
        grid=(nb,),
        in_specs=[
            smem,
            smem,
            pl.BlockSpec((b, _W), lambda j: (0, j)),
        ],
        out_specs=pl.BlockSpec((b, _W), lambda j: (0, j)),
        out_shape=jax.ShapeDtypeStruct((b, t), clean_waveform.dtype),
        compiler_params=pltpu.CompilerParams(
            dimension_semantics=("arbitrary",),
        ),
    )(jnp.asarray(ent), jnp.asarray(cnt), clean_waveform)
